# baseline (speedup 1.0000x reference)
import functools

import jax
import jax.numpy as jnp
from jax.experimental import pallas as pl
from jax.experimental.pallas import tpu as pltpu
from jax.experimental.pallas import tpu_sc as plsc

B = 128
IN = 64
N = 32768
K = 8
BLK = 2048
NB = N // BLK
NG = 16
NW = 32
RPW = B // NW
IMAX = 2**31 - 1


def _tc_pass_a(xt_ref, wt_ref, b_ref, y_ref, d_ref, r_ref):
    j = pl.program_id(0)
    xtb = xt_ref[...]
    wtb = wt_ref[...]
    yb = jax.lax.dot_general(xtb, wtb, (((0,), (0,)), ((), ())),
                             preferred_element_type=jnp.float32)
    y_ref[...] = yb + b_ref[...]
    x2 = jnp.sum(xtb * xtb, axis=0)[:, None]
    w2 = jnp.sum(wtb * wtb, axis=0)[None, :]
    s = jnp.sqrt(jnp.maximum(x2 + w2 - 2.0 * yb, 0.0))
    for u in range(BLK // 128):
        d_ref[:, u, :, :] = s[:, u * 128:(u + 1) * 128].reshape(B // 8, 8, 128)

    @pl.when(j == 0)
    def _():
        r_ref[...] = s

    @pl.when(j > 0)
    def _():
        r_ref[...] = jnp.maximum(r_ref[...], s)


def _tc_pass_b(r_ref, g_ref):
    r = jnp.maximum(r_ref[:, :BLK // 2], r_ref[:, BLK // 2:])
    lane = jax.lax.broadcasted_iota(jnp.int32, (B, BLK // 2), 1)
    gs = []
    for _ in range(12):
        m = jnp.max(r, axis=1, keepdims=True)
        g = jnp.min(jnp.where(r == m, lane, IMAX), axis=1, keepdims=True)
        gs.append(g)
        r = jnp.where(lane == g, -1.0, r)
    gs += [gs[0]] * (NG - 12)
    g_ref[...] = jnp.concatenate(gs, axis=1)


_sc_mesh = plsc.VectorSubcoreMesh(core_axis_name="c", subcore_axis_name="s")


@functools.partial(
    pl.kernel,
    out_type=jax.ShapeDtypeStruct((B, N), jnp.float32),
    mesh=_sc_mesh,
    scratch_types=[
        pltpu.VMEM((N,), jnp.float32),
        pltpu.VMEM((N,), jnp.float32),
        pltpu.VMEM((RPW, NG), jnp.int32),
        pltpu.VMEM((8, 128), jnp.int32),
        pltpu.VMEM((8, 128), jnp.float32),
        pltpu.SemaphoreType.DMA,
        pltpu.SemaphoreType.DMA,
        pltpu.SemaphoreType.DMA,
        pltpu.SemaphoreType.DMA,
    ],
    compiler_params=pltpu.CompilerParams(needs_layout_passes=False),
)
def _sc_scatter(d_hbm, g_hbm, wta_hbm, rowbuf0, rowbuf1, gbuf, idxbuf,
                candbuf, gsem0, gsem1, wsem0, wsem1):
    rowbufs = (rowbuf0, rowbuf1)
    wid = jax.lax.axis_index("s") * 2 + jax.lax.axis_index("c")
    lane = jax.lax.iota(jnp.int32, 16)
    zeros = jnp.zeros((16,), jnp.float32)
    ones = jnp.full((16,), 1.0, jnp.float32)
    wsems = (wsem0, wsem1)

    _dn = jax.lax.GatherDimensionNumbers(
        offset_dims=(), collapsed_slice_dims=(0,), start_index_map=(0,))

    def _shuf(v, perm):
        return jax.lax.gather(
            v, perm[:, None], _dn, slice_sizes=(1,),
            mode=jax.lax.GatherScatterMode.PROMISE_IN_BOUNDS)

    def _allmax(v):
        for sh in (8, 4, 2, 1):
            v = jnp.maximum(v, _shuf(v, (lane + sh) & 15))
        return v

    def _allmin(v):
        for sh in (8, 4, 2, 1):
            v = jnp.minimum(v, _shuf(v, (lane + sh) & 15))
        return v

    pltpu.sync_copy(g_hbm.at[pl.ds(wid * RPW, RPW)], gbuf)
    gsems = (gsem0, gsem1)
    NM = 2 * NB

    def _start_gather(t):
        r = wid * RPW + t
        q = t & 1
        g = gbuf[t]
        base = (r >> 3) * (N * 8) + (r & 7) * 128
        ghi = (g >> 7) * 1024
        glo = g & 127
        cps = []
        for m in range(NM):
            idxbuf[q * 4 + m // 8, pl.ds((m % 8) * 16, 16)] = (
                base + m * 8192 + ghi + glo)
        for u in range(4):
            cps.append(pltpu.async_copy(d_hbm.at[idxbuf.at[q * 4 + u]],
                                        candbuf.at[q * 4 + u], gsems[q]))
        return cps

    def _zero(buf):
        def _z(i, _):
            for u in range(8):
                buf[pl.ds((i * 8 + u) * 16, 16)] = zeros
            return 0
        jax.lax.fori_loop(0, N // (16 * 8), _z, 0)

    prev = [None, None]
    gcp = _start_gather(0)
    _zero(rowbuf0)
    for t in range(RPW):
        p = t & 1
        r = wid * RPW + t
        g = gbuf[t]
        for cp in gcp:
            cp.wait()
        cvals = [candbuf[p * 4 + m // 8, pl.ds((m % 8) * 16, 16)]
                 for m in range(NM)]
        if t + 1 < RPW:
            gcp = _start_gather(t + 1)
        ridx = [g + m * (BLK // 2) for m in range(NM)]
        winv = None
        for k in range(K):
            bv, bi = cvals[0], ridx[0]
            for i in range(1, NM):
                better = (cvals[i] > bv) | ((cvals[i] == bv) & (ridx[i] < bi))
                bv = jnp.where(better, cvals[i], bv)
                bi = jnp.where(better, ridx[i], bi)
            m = _allmax(bv)
            win = _allmin(jnp.where(bv == m, bi, IMAX))
            winv = win if winv is None else jnp.where(lane == k, win, winv)
            for i in range(NM):
                cvals[i] = jnp.where(ridx[i] == win, -1.0, cvals[i])
        if prev[p] is not None:
            prev[p][0].wait()
            plsc.store_scatter(rowbufs[p], [prev[p][1]], zeros)
        plsc.store_scatter(rowbufs[p], [winv], ones)
        cp = pltpu.async_copy(rowbufs[p], wta_hbm.at[r], wsems[p])
        prev[p] = (cp, winv)
        if t == 0:
            _zero(rowbuf1)
    for p in range(2):
        if prev[p] is not None:
            prev[p][0].wait()


def kernel(x, W, b):
    b2 = b.reshape(1, N)
    y, dist, R = pl.pallas_call(
        _tc_pass_a,
        grid=(NB,),
        in_specs=[
            pl.BlockSpec((IN, B), lambda j: (0, 0)),
            pl.BlockSpec((IN, BLK), lambda j: (0, j)),
            pl.BlockSpec((1, BLK), lambda j: (0, j)),
        ],
        out_specs=[
            pl.BlockSpec((B, BLK), lambda j: (0, j)),
            pl.BlockSpec((B // 8, BLK // 128, 8, 128), lambda j: (0, j, 0, 0)),
            pl.BlockSpec((B, BLK), lambda j: (0, 0)),
        ],
        out_shape=[
            jax.ShapeDtypeStruct((B, N), jnp.float32),
            jax.ShapeDtypeStruct((B // 8, N // 128, 8, 128), jnp.float32),
            jax.ShapeDtypeStruct((B, BLK), jnp.float32),
        ],
    )(x.T, W.T, b2)
    G = pl.pallas_call(
        _tc_pass_b,
        in_specs=[pl.BlockSpec((B, BLK), lambda: (0, 0))],
        out_specs=pl.BlockSpec((B, NG), lambda: (0, 0)),
        out_shape=jax.ShapeDtypeStruct((B, NG), jnp.int32),
    )(R)
    wta = _sc_scatter(dist.reshape(B * N), G)
    return (y, wta)


# speedup vs baseline: 1.1085x; 1.1085x over previous
import functools

import jax
import jax.numpy as jnp
from jax.experimental import pallas as pl
from jax.experimental.pallas import tpu as pltpu
from jax.experimental.pallas import tpu_sc as plsc

B = 128
IN = 64
N = 32768
K = 8
BLK = 2048
NB = N // BLK
NG = 16
NW = 32
RPW = B // NW
IMAX = 2**31 - 1


def _tc_pass_a(xt_ref, wt_ref, b_ref, y_ref, d_ref, r_ref):
    j = pl.program_id(0)
    xtb = xt_ref[...]
    wtb = wt_ref[...]
    yb = jax.lax.dot_general(xtb, wtb, (((0,), (0,)), ((), ())),
                             preferred_element_type=jnp.float32)
    y_ref[...] = yb + b_ref[...]
    x2 = jnp.sum(xtb * xtb, axis=0)[:, None]
    w2 = jnp.sum(wtb * wtb, axis=0)[None, :]
    s = jnp.sqrt(jnp.maximum(x2 + w2 - 2.0 * yb, 0.0))
    for u in range(BLK // 128):
        d_ref[:, u, :, :] = s[:, u * 128:(u + 1) * 128].reshape(B // 8, 8, 128)

    @pl.when(j == 0)
    def _():
        r_ref[...] = s

    @pl.when(j > 0)
    def _():
        r_ref[...] = jnp.maximum(r_ref[...], s)


def _tc_pass_b(r_ref, g_ref):
    r = r_ref[...]
    lane = jax.lax.broadcasted_iota(jnp.int32, (B, BLK), 1)
    gs = []
    for _ in range(12):
        m = jnp.max(r, axis=1, keepdims=True)
        g = jnp.min(jnp.where(r == m, lane, IMAX), axis=1, keepdims=True)
        gs.append(g)
        r = jnp.where(lane == g, -1.0, r)
    gs += [gs[0]] * (NG - 12)
    g_ref[...] = jnp.concatenate(gs, axis=1)


_sc_mesh = plsc.VectorSubcoreMesh(core_axis_name="c", subcore_axis_name="s")


@functools.partial(
    pl.kernel,
    out_type=jax.ShapeDtypeStruct((B, N), jnp.float32),
    mesh=_sc_mesh,
    scratch_types=[
        pltpu.VMEM((N,), jnp.float32),
        pltpu.VMEM((N,), jnp.float32),
        pltpu.VMEM((RPW, NG), jnp.int32),
        pltpu.VMEM((8, 128), jnp.int32),
        pltpu.VMEM((8, 128), jnp.float32),
        pltpu.SemaphoreType.DMA,
        pltpu.SemaphoreType.DMA,
        pltpu.SemaphoreType.DMA,
        pltpu.SemaphoreType.DMA,
    ],
    compiler_params=pltpu.CompilerParams(needs_layout_passes=False),
)
def _sc_scatter(d_hbm, g_hbm, wta_hbm, rowbuf0, rowbuf1, gbuf, idxbuf,
                candbuf, gsem0, gsem1, wsem0, wsem1):
    rowbufs = (rowbuf0, rowbuf1)
    wid = jax.lax.axis_index("s") * 2 + jax.lax.axis_index("c")
    lane = jax.lax.iota(jnp.int32, 16)
    zeros = jnp.zeros((16,), jnp.float32)
    ones = jnp.full((16,), 1.0, jnp.float32)
    wsems = (wsem0, wsem1)

    _dn = jax.lax.GatherDimensionNumbers(
        offset_dims=(), collapsed_slice_dims=(0,), start_index_map=(0,))

    def _shuf(v, perm):
        return jax.lax.gather(
            v, perm[:, None], _dn, slice_sizes=(1,),
            mode=jax.lax.GatherScatterMode.PROMISE_IN_BOUNDS)

    def _allmax(v):
        for sh in (8, 4, 2, 1):
            v = jnp.maximum(v, _shuf(v, (lane + sh) & 15))
        return v

    def _allmin(v):
        for sh in (8, 4, 2, 1):
            v = jnp.minimum(v, _shuf(v, (lane + sh) & 15))
        return v

    pltpu.sync_copy(g_hbm.at[pl.ds(wid * RPW, RPW)], gbuf)
    gsems = (gsem0, gsem1)
    NM = NB

    def _start_gather(t):
        r = wid * RPW + t
        q = t & 1
        g = gbuf[t]
        base = (r >> 3) * (N * 8) + (r & 7) * 128
        ghi = (g >> 7) * 1024
        glo = g & 127
        cps = []
        for m in range(NM):
            idxbuf[q * 2 + m // 8, pl.ds((m % 8) * 16, 16)] = (
                base + m * 16384 + ghi + glo)
        for u in range(2):
            cps.append(pltpu.async_copy(d_hbm.at[idxbuf.at[q * 2 + u]],
                                        candbuf.at[q * 2 + u], gsems[q]))
        return cps

    def _zero(buf):
        def _z(i, _):
            for u in range(8):
                buf[pl.ds((i * 8 + u) * 16, 16)] = zeros
            return 0
        jax.lax.fori_loop(0, N // (16 * 8), _z, 0)

    prev = [None, None]
    gcp = _start_gather(0)
    _zero(rowbuf0)
    for t in range(RPW):
        p = t & 1
        r = wid * RPW + t
        g = gbuf[t]
        for cp in gcp:
            cp.wait()
        cvals = [candbuf[p * 2 + m // 8, pl.ds((m % 8) * 16, 16)]
                 for m in range(NM)]
        if t + 1 < RPW:
            gcp = _start_gather(t + 1)
        ridx = [g + m * BLK for m in range(NM)]
        winv = None
        for k in range(K):
            bv, bi = cvals[0], ridx[0]
            for i in range(1, NM):
                better = (cvals[i] > bv) | ((cvals[i] == bv) & (ridx[i] < bi))
                bv = jnp.where(better, cvals[i], bv)
                bi = jnp.where(better, ridx[i], bi)
            m = _allmax(bv)
            win = _allmin(jnp.where(bv == m, bi, IMAX))
            winv = win if winv is None else jnp.where(lane == k, win, winv)
            for i in range(NM):
                cvals[i] = jnp.where(ridx[i] == win, -1.0, cvals[i])
        if prev[p] is not None:
            prev[p][0].wait()
            plsc.store_scatter(rowbufs[p], [prev[p][1]], zeros)
        plsc.store_scatter(rowbufs[p], [winv], ones)
        cp = pltpu.async_copy(rowbufs[p], wta_hbm.at[r], wsems[p])
        prev[p] = (cp, winv)
        if t == 0:
            _zero(rowbuf1)
    for p in range(2):
        if prev[p] is not None:
            prev[p][0].wait()


def kernel(x, W, b):
    b2 = b.reshape(1, N)
    y, dist, R = pl.pallas_call(
        _tc_pass_a,
        grid=(NB,),
        in_specs=[
            pl.BlockSpec((IN, B), lambda j: (0, 0)),
            pl.BlockSpec((IN, BLK), lambda j: (0, j)),
            pl.BlockSpec((1, BLK), lambda j: (0, j)),
        ],
        out_specs=[
            pl.BlockSpec((B, BLK), lambda j: (0, j)),
            pl.BlockSpec((B // 8, BLK // 128, 8, 128), lambda j: (0, j, 0, 0)),
            pl.BlockSpec((B, BLK), lambda j: (0, 0)),
        ],
        out_shape=[
            jax.ShapeDtypeStruct((B, N), jnp.float32),
            jax.ShapeDtypeStruct((B // 8, N // 128, 8, 128), jnp.float32),
            jax.ShapeDtypeStruct((B, BLK), jnp.float32),
        ],
    )(x.T, W.T, b2)
    G = pl.pallas_call(
        _tc_pass_b,
        in_specs=[pl.BlockSpec((B, BLK), lambda: (0, 0))],
        out_specs=pl.BlockSpec((B, NG), lambda: (0, 0)),
        out_shape=jax.ShapeDtypeStruct((B, NG), jnp.int32),
    )(R)
    wta = _sc_scatter(dist.reshape(B * N), G)
    return (y, wta)
